# baseline (speedup 1.0000x reference)
import functools

import jax
import jax.numpy as jnp
from jax import lax
from jax.experimental import pallas as pl
from jax.experimental.pallas import tpu as pltpu

_K = 8
_BIG = 2**30
_ROWS = 8


def _scan_block(src_ref, base, t_tile, p_tile):
    iota = lax.broadcasted_iota(jnp.int32, (_ROWS, p_tile), 1)
    bvs = []
    bis = []
    for g in range(t_tile // _ROWS):
        c = src_ref[g * _ROWS:(g + 1) * _ROWS, :]
        ms = []
        ps = []
        for _ in range(_K):
            m = jnp.max(c, axis=1, keepdims=True)
            pos = jnp.min(jnp.where(c == m, iota, _BIG), axis=1,
                          keepdims=True)
            c = jnp.where(iota == pos, -jnp.inf, c)
            ms.append(m)
            ps.append(pos)
        bvs.append(jnp.concatenate(ms, axis=1))
        bis.append(jnp.concatenate(ps, axis=1))
    bv = jnp.concatenate(bvs, axis=0)
    bi = base + jnp.concatenate(bis, axis=0)
    return bv, bi


def _merge_runs(bv, bi, vals_scr, gidx_scr):
    cv = jnp.concatenate([vals_scr[...], bv], axis=1)
    ci = jnp.concatenate([gidx_scr[...], bi], axis=1)
    nv = []
    ni = []
    for _ in range(_K):
        m = jnp.max(cv, axis=1, keepdims=True)
        gi = jnp.min(jnp.where(cv == m, ci, _BIG), axis=1, keepdims=True)
        cv = jnp.where(ci == gi, -jnp.inf, cv)
        nv.append(m)
        ni.append(gi)
    vals_scr[...] = jnp.concatenate(nv, axis=1)
    gidx_scr[...] = jnp.concatenate(ni, axis=1)


def _router_body(x_ref, w_ref, b_ref, idx_ref, sc_ref, vals_scr, gidx_scr,
                 sprev_scr):
    j = pl.program_id(1)
    nj = pl.num_programs(1)
    t_tile, _ = x_ref.shape
    p_tile = w_ref.shape[0]

    s = jax.lax.dot_general(
        x_ref[...], w_ref[...], (((1,), (1,)), ((), ())),
        preferred_element_type=jnp.float32,
        precision=jax.lax.Precision.DEFAULT,
    )
    s = s + b_ref[pl.ds(j * p_tile, p_tile)][None, :]
    sc_ref[...] = s

    @pl.when(j == 0)
    def _init():
        vals_scr[...] = jnp.full_like(vals_scr, -jnp.inf)
        gidx_scr[...] = jnp.zeros_like(gidx_scr)
        sprev_scr[...] = jnp.full_like(sprev_scr, -jnp.inf)

    bv, bi = _scan_block(sprev_scr, (j - 1) * p_tile, t_tile, p_tile)
    _merge_runs(bv, bi, vals_scr, gidx_scr)

    sprev_scr[...] = s

    @pl.when(j == nj - 1)
    def _fin():
        bv2, bi2 = _scan_block(sprev_scr, j * p_tile, t_tile, p_tile)
        _merge_runs(bv2, bi2, vals_scr, gidx_scr)
        idx_ref[...] = gidx_scr[...]


@functools.partial(jax.jit, static_argnames=("interpret",))
def _router(x2d, w, b, interpret=False):
    t, d = x2d.shape
    p = w.shape[0]
    t_tile = min(512, t)
    p_tile = min(512, p)
    grid = (t // t_tile, p // p_tile)
    idx_out, scores = pl.pallas_call(
        _router_body,
        grid=grid,
        in_specs=[
            pl.BlockSpec((t_tile, d), lambda i, j: (i, 0)),
            pl.BlockSpec((p_tile, d), lambda i, j: (j, 0)),
            pl.BlockSpec((p,), lambda i, j: (0,)),
        ],
        out_specs=[
            pl.BlockSpec((t_tile, _K), lambda i, j: (i, 0)),
            pl.BlockSpec((t_tile, p_tile), lambda i, j: (i, j)),
        ],
        out_shape=[
            jax.ShapeDtypeStruct((t, _K), jnp.int32),
            jax.ShapeDtypeStruct((t, p), jnp.float32),
        ],
        scratch_shapes=[
            pltpu.VMEM((t_tile, _K), jnp.float32),
            pltpu.VMEM((t_tile, _K), jnp.int32),
            pltpu.VMEM((t_tile, p_tile), jnp.float32),
        ],
        compiler_params=pltpu.CompilerParams(
            dimension_semantics=("parallel", "arbitrary"),
        ),
        interpret=interpret,
    )(x2d, w, b)
    return idx_out, scores


def kernel(x, w, b):
    bsz, seq, d = x.shape
    p = w.shape[0]
    x2d = x.reshape(bsz * seq, d)
    idx_out, scores = _router(x2d, w, b)
    return idx_out.reshape(bsz, seq, _K), scores.reshape(bsz, seq, p)


# speedup vs baseline: 1.6917x; 1.6917x over previous
import functools

import jax
import jax.numpy as jnp
from jax import lax
from jax.experimental import pallas as pl
from jax.experimental.pallas import tpu as pltpu
from jax.experimental.pallas import tpu_sc as plsc

_K = 8
_NEG = float("-inf")


def _scores_body(x_ref, w_ref, b_ref, sc_ref):
    j = pl.program_id(1)
    p_tile = w_ref.shape[0]
    s = jax.lax.dot_general(
        x_ref[...], w_ref[...], (((1,), (1,)), ((), ())),
        preferred_element_type=jnp.float32,
        precision=jax.lax.Precision.DEFAULT,
    )
    sc_ref[...] = s + b_ref[pl.ds(j * p_tile, p_tile)][None, :]


def _tc_scores(x2d, w, b):
    t, d = x2d.shape
    p = w.shape[0]
    t_tile = min(512, t)
    p_tile = min(512, p)
    return pl.pallas_call(
        _scores_body,
        grid=(t // t_tile, p // p_tile),
        in_specs=[
            pl.BlockSpec((t_tile, d), lambda i, j: (i, 0)),
            pl.BlockSpec((p_tile, d), lambda i, j: (j, 0)),
            pl.BlockSpec((p,), lambda i, j: (0,)),
        ],
        out_specs=pl.BlockSpec((t_tile, p_tile), lambda i, j: (i, j)),
        out_shape=jax.ShapeDtypeStruct((t, p), jnp.float32),
        compiler_params=pltpu.CompilerParams(
            dimension_semantics=("parallel", "arbitrary"),
        ),
    )(x2d, w, b)


_SUPER = 16


def _sc_topk(scores):
    t, p = scores.shape
    info = plsc.get_sparse_core_info()
    nc = info.num_cores
    nw = nc * info.num_subcores
    rows_w = t // nw
    n_super = p // (16 * _SUPER)

    mesh = plsc.VectorSubcoreMesh(core_axis_name="c", subcore_axis_name="s")

    @functools.partial(
        pl.kernel, mesh=mesh,
        compiler_params=pltpu.CompilerParams(needs_layout_passes=False),
        out_type=jax.ShapeDtypeStruct((t * _K,), jnp.int32),
        scratch_types=[
            pltpu.VMEM((p,), jnp.float32),
            pltpu.VMEM((p,), jnp.float32),
            pltpu.VMEM((rows_w * _K + 16,), jnp.int32),
            pltpu.SemaphoreType.DMA,
            pltpu.SemaphoreType.DMA,
        ],
    )
    def k(scores_hbm, idx_hbm, buf0, buf1, outbuf, sem0, sem1):
        wid = lax.axis_index("s") * nc + lax.axis_index("c")
        base = wid * rows_w
        lane = lax.broadcasted_iota(jnp.int32, (16,), 0)

        def merge_sorted(sv, si, carry):
            cv, ci, _ = carry
            sel = cv >= sv
            nv = jnp.where(sel, cv, sv)
            ni = jnp.where(sel, ci, si)
            cv, ci = plsc.sort_key_val(nv, ni, descending=False)
            return cv, ci, cv[0]

        def scan_row(buf, row_local):
            def sc_body(q, carry):
                c = carry
                for u in range(_SUPER):
                    col0 = q * (16 * _SUPER) + u * 16
                    v = buf[pl.ds(col0, 16)]
                    sv, si = plsc.sort_key_val(v, col0 + lane,
                                               descending=True)
                    c = merge_sorted(sv, si, c)
                return c

            init = (jnp.full((16,), _NEG, jnp.float32),
                    jnp.zeros((16,), jnp.int32),
                    jnp.float32(_NEG))
            cv, ci, _ = lax.fori_loop(0, n_super, sc_body, init)
            sv, si = plsc.sort_key_val(cv, ci, descending=True)
            outbuf[pl.ds(row_local * _K, 16)] = si

        pltpu.async_copy(scores_hbm.at[base], buf0, sem0)

        def pair_body(i, _):
            row_e = 2 * i
            row_o = row_e + 1
            pltpu.async_copy(scores_hbm.at[base + row_o], buf1, sem1)
            pltpu.make_async_copy(scores_hbm.at[base + row_e], buf0,
                                  sem0).wait()
            scan_row(buf0, row_e)

            @pl.when(row_e + 2 < rows_w)
            def _():
                pltpu.async_copy(scores_hbm.at[base + row_e + 2], buf0, sem0)

            pltpu.make_async_copy(scores_hbm.at[base + row_o], buf1,
                                  sem1).wait()
            scan_row(buf1, row_o)

            @pl.when(row_o + 2 < rows_w)
            def _():
                pltpu.async_copy(scores_hbm.at[base + row_o + 2], buf1, sem1)

            return 0

        lax.fori_loop(0, rows_w // 2, pair_body, 0)
        pltpu.sync_copy(outbuf.at[pl.ds(0, rows_w * _K)],
                        idx_hbm.at[pl.ds(base * _K, rows_w * _K)])

    return k(scores).reshape(t, _K)


@jax.jit
def _router(x2d, w, b):
    scores = _tc_scores(x2d, w, b)
    idx = _sc_topk(scores)
    return idx, scores


def kernel(x, w, b):
    bsz, seq, d = x.shape
    p = w.shape[0]
    x2d = x.reshape(bsz * seq, d)
    idx_out, scores = _router(x2d, w, b)
    return idx_out.reshape(bsz, seq, _K), scores.reshape(bsz, seq, p)
